# baseline (speedup 1.0000x reference)
import jax
import jax.numpy as jnp
from jax.experimental import pallas as pl

_BM = 400


def _xw_kernel(x_ref, w_ref, o_ref):
    o_ref[...] = jnp.dot(x_ref[...], w_ref[...],
                         preferred_element_type=jnp.float32)


def _layer1_kernel(adj_ref, s_ref, b1_ref, w2_ref, o_ref):
    z = jnp.dot(adj_ref[...], s_ref[...],
                preferred_element_type=jnp.float32) + b1_ref[...]
    h = jnp.maximum(z, 0.0)
    o_ref[...] = jnp.dot(h, w2_ref[...], preferred_element_type=jnp.float32)


def _layer2_kernel(adj_ref, t_ref, b2_ref, o_ref):
    z = jnp.dot(adj_ref[...], t_ref[...],
                preferred_element_type=jnp.float32) + b2_ref[...]
    m = jnp.max(z, axis=1, keepdims=True)
    e = jnp.exp(z - m)
    lse = jnp.log(jnp.sum(e, axis=1, keepdims=True))
    o_ref[...] = z - m - lse


def kernel(x, adj, W1, b1, W2, b2):
    n, nfeat = x.shape
    nhid = W1.shape[1]
    ncls = W2.shape[1]
    b1r = b1.reshape(1, nhid)
    b2r = b2.reshape(1, ncls)

    s1 = pl.pallas_call(
        _xw_kernel,
        out_shape=jax.ShapeDtypeStruct((n, nhid), jnp.float32),
    )(x, W1)

    grid = (n // _BM,)
    adj_spec = pl.BlockSpec((_BM, n), lambda i: (i, 0))

    t = pl.pallas_call(
        _layer1_kernel,
        grid=grid,
        in_specs=[
            adj_spec,
            pl.BlockSpec((n, nhid), lambda i: (0, 0)),
            pl.BlockSpec((1, nhid), lambda i: (0, 0)),
            pl.BlockSpec((nhid, ncls), lambda i: (0, 0)),
        ],
        out_specs=pl.BlockSpec((_BM, ncls), lambda i: (i, 0)),
        out_shape=jax.ShapeDtypeStruct((n, ncls), jnp.float32),
    )(adj, s1, b1r, W2)

    out = pl.pallas_call(
        _layer2_kernel,
        grid=grid,
        in_specs=[
            adj_spec,
            pl.BlockSpec((n, ncls), lambda i: (0, 0)),
            pl.BlockSpec((1, ncls), lambda i: (0, 0)),
        ],
        out_specs=pl.BlockSpec((_BM, ncls), lambda i: (i, 0)),
        out_shape=jax.ShapeDtypeStruct((n, ncls), jnp.float32),
    )(adj, t, b2r)
    return out


# speedup vs baseline: 1.0427x; 1.0427x over previous
import jax
import jax.numpy as jnp
from jax.experimental import pallas as pl
from jax.experimental.pallas import tpu as pltpu

_BM = 400


def _fused_kernel(x_ref, adj_ref, w1_ref, b1_ref, w2_ref, b2_ref,
                  o_ref, s1_ref, t_ref, *, nb, bm):
    i = pl.program_id(0)

    @pl.when(i == 0)
    def _():
        s1_ref[...] = jnp.dot(x_ref[...], w1_ref[...],
                              preferred_element_type=jnp.float32)

    @pl.when((i >= 1) & (i <= nb))
    def _():
        z = jnp.dot(adj_ref[...], s1_ref[...],
                    preferred_element_type=jnp.float32) + b1_ref[...]
        h = jnp.maximum(z, 0.0)
        t_ref[pl.ds((i - 1) * bm, bm), :] = jnp.dot(
            h, w2_ref[...], preferred_element_type=jnp.float32)

    @pl.when(i > nb)
    def _():
        z = jnp.dot(adj_ref[...], t_ref[...],
                    preferred_element_type=jnp.float32) + b2_ref[...]
        m = jnp.max(z, axis=1, keepdims=True)
        e = jnp.exp(z - m)
        lse = jnp.log(jnp.sum(e, axis=1, keepdims=True))
        o_ref[...] = z - m - lse


def kernel(x, adj, W1, b1, W2, b2):
    import functools
    n, nfeat = x.shape
    nhid = W1.shape[1]
    ncls = W2.shape[1]
    nb = n // _BM
    b1r = b1.reshape(1, nhid)
    b2r = b2.reshape(1, ncls)

    adj_map = lambda i: (jnp.where(i == 0, 0, (i - 1) % nb), 0)
    out_map = lambda i: (jnp.maximum(i - nb - 1, 0), 0)

    return pl.pallas_call(
        functools.partial(_fused_kernel, nb=nb, bm=_BM),
        grid=(2 * nb + 1,),
        in_specs=[
            pl.BlockSpec((n, nfeat), lambda i: (0, 0)),
            pl.BlockSpec((_BM, n), adj_map),
            pl.BlockSpec((nfeat, nhid), lambda i: (0, 0)),
            pl.BlockSpec((1, nhid), lambda i: (0, 0)),
            pl.BlockSpec((nhid, ncls), lambda i: (0, 0)),
            pl.BlockSpec((1, ncls), lambda i: (0, 0)),
        ],
        out_specs=pl.BlockSpec((_BM, ncls), out_map),
        out_shape=jax.ShapeDtypeStruct((n, ncls), jnp.float32),
        scratch_shapes=[
            pltpu.VMEM((n, nhid), jnp.float32),
            pltpu.VMEM((n, ncls), jnp.float32),
        ],
    )(x, adj, W1, b1r, W2, b2r)
